# baseline (device time: 98952 ns/iter reference)
import jax
import jax.numpy as jnp
from jax import lax
from jax.experimental import pallas as pl
from jax.experimental.pallas import tpu as pltpu

N_DEV = 16
N_STREAMS = 4

RING = (0, 4, 8, 12, 15, 11, 7, 3, 2, 6, 10, 14, 13, 9, 5, 1)
POS = tuple(RING.index(i) for i in range(N_DEV))


def _lut(table, idx):
    out = jnp.int32(table[0])
    for j in range(1, len(table)):
        out = jnp.where(idx == j, jnp.int32(table[j]), out)
    return out


def kernel(x, w_mat):
    m, k_shard = x.shape
    _, n = w_mat.shape
    m_blk = m // N_DEV
    n_half = n // 2
    n_sub = n_half // N_STREAMS

    def body(x_ref, w_ref, out_ref, wb_ref, *comm_and_sems):
        comms = comm_and_sems[:2 * N_STREAMS]
        semss = comm_and_sems[2 * N_STREAMS:]

        d = lax.axis_index("i")
        r_pos = _lut(POS, d)
        right = _lut(RING, lax.rem(r_pos + 1, N_DEV))
        left = _lut(RING, lax.rem(r_pos + N_DEV - 1, N_DEV))

        barrier_sem = pltpu.get_barrier_semaphore()
        pl.semaphore_signal(barrier_sem, inc=1, device_id=(left,),
                            device_id_type=pl.DeviceIdType.MESH)
        pl.semaphore_signal(barrier_sem, inc=1, device_id=(right,),
                            device_id_type=pl.DeviceIdType.MESH)
        wb_ref[...] = w_ref[...].astype(jnp.bfloat16)
        pl.semaphore_wait(barrier_sem, 2)

        def partial(c, half):
            xa = x_ref[pl.ds(c * m_blk, m_blk), :].astype(jnp.bfloat16)
            wh = wb_ref[:, half * n_half:(half + 1) * n_half]
            return jnp.dot(xa, wh, preferred_element_type=jnp.float32)

        def sub_partial(c, half, off):
            xa = x_ref[pl.ds(c * m_blk, m_blk), :].astype(jnp.bfloat16)
            lo = half * n_half + off
            return jnp.dot(xa, wb_ref[:, lo:lo + n_sub],
                           preferred_element_type=jnp.float32)

        def chunk_r(s):
            return _lut(RING, lax.rem(r_pos + 2 * N_DEV - 2 - s, N_DEV))

        def chunk_l(s):
            return _lut(RING, lax.rem(r_pos + 2 + s, N_DEV))

        def mk(comm, sems, s, dst):
            return pltpu.make_async_remote_copy(
                src_ref=comm.at[s], dst_ref=comm.at[s + 1],
                send_sem=sems.at[s, 0], recv_sem=sems.at[s, 1],
                device_id=(dst,), device_id_type=pl.DeviceIdType.MESH,
            )

        streams = []
        for k in range(2 * N_STREAMS):
            half = k // N_STREAMS
            off = (k % N_STREAMS) * n_sub
            dst = right if half == 0 else left
            streams.append((comms[k], semss[k], dst, half, off))

        for comm, sems, dst, half, off in streams:
            c0 = left if half == 0 else right
            comm[0] = sub_partial(c0, half, off).astype(jnp.bfloat16)
            mk(comm, sems, 0, dst).start()

        for s in range(N_DEV - 1):
            last = s == N_DEV - 2
            p_r = partial(chunk_r(s), 0)
            p_l = partial(chunk_l(s), 1)
            for comm, sems, dst, half, off in streams:
                p = p_r if half == 0 else p_l
                mk(comm, sems, s, dst).wait()
                acc = comm[s + 1].astype(jnp.float32) + p[:, off:off + n_sub]
                if not last:
                    comm[s + 1] = acc.astype(jnp.bfloat16)
                    mk(comm, sems, s + 1, dst).start()
                else:
                    col = half * n_half + off
                    out_ref[:, col:col + n_sub] = acc * jax.nn.sigmoid(acc)

        def exit_barrier(sem):
            pl.semaphore_signal(sem, inc=1, device_id=(left,),
                                device_id_type=pl.DeviceIdType.MESH)
            pl.semaphore_signal(sem, inc=1, device_id=(right,),
                                device_id_type=pl.DeviceIdType.MESH)
            pl.semaphore_wait(sem, 2)

        pl.run_scoped(exit_barrier, pltpu.SemaphoreType.REGULAR)

    sub = pltpu.VMEM((N_DEV, m_blk, n_sub), jnp.bfloat16)
    sems = pltpu.SemaphoreType.DMA((N_DEV - 1, 2))
    return pl.pallas_call(
        body,
        out_shape=jax.ShapeDtypeStruct((m_blk, n), jnp.float32),
        in_specs=[
            pl.BlockSpec(memory_space=pltpu.VMEM),
            pl.BlockSpec(memory_space=pltpu.VMEM),
        ],
        out_specs=pl.BlockSpec(memory_space=pltpu.VMEM),
        scratch_shapes=(
            [pltpu.VMEM((k_shard, n), jnp.bfloat16)]
            + [sub] * (2 * N_STREAMS)
            + [sems] * (2 * N_STREAMS)
        ),
        compiler_params=pltpu.CompilerParams(collective_id=0),
    )(x, w_mat)


# device time: 98091 ns/iter; 1.0088x vs baseline; 1.0088x over previous
import jax
import jax.numpy as jnp
from jax import lax
from jax.experimental import pallas as pl
from jax.experimental.pallas import tpu as pltpu

N_DEV = 16
N_STREAMS = 2

RING = (0, 4, 8, 12, 15, 11, 7, 3, 2, 6, 10, 14, 13, 9, 5, 1)
POS = tuple(RING.index(i) for i in range(N_DEV))


def _lut(table, idx):
    out = jnp.int32(table[0])
    for j in range(1, len(table)):
        out = jnp.where(idx == j, jnp.int32(table[j]), out)
    return out


def kernel(x, w_mat):
    m, k_shard = x.shape
    _, n = w_mat.shape
    m_blk = m // N_DEV
    n_half = n // 2
    n_sub = n_half // N_STREAMS

    def body(x_ref, w_ref, out_ref, wb_ref, *comm_and_sems):
        comms = comm_and_sems[:2 * N_STREAMS]
        semss = comm_and_sems[2 * N_STREAMS:]

        d = lax.axis_index("i")
        r_pos = _lut(POS, d)
        right = _lut(RING, lax.rem(r_pos + 1, N_DEV))
        left = _lut(RING, lax.rem(r_pos + N_DEV - 1, N_DEV))

        barrier_sem = pltpu.get_barrier_semaphore()
        pl.semaphore_signal(barrier_sem, inc=1, device_id=(left,),
                            device_id_type=pl.DeviceIdType.MESH)
        pl.semaphore_signal(barrier_sem, inc=1, device_id=(right,),
                            device_id_type=pl.DeviceIdType.MESH)
        wb_ref[...] = w_ref[...].astype(jnp.bfloat16)
        pl.semaphore_wait(barrier_sem, 2)

        def partial(c, half):
            xa = x_ref[pl.ds(c * m_blk, m_blk), :].astype(jnp.bfloat16)
            wh = wb_ref[:, half * n_half:(half + 1) * n_half]
            return jnp.dot(xa, wh, preferred_element_type=jnp.float32)

        def sub_partial(c, half, off):
            xa = x_ref[pl.ds(c * m_blk, m_blk), :].astype(jnp.bfloat16)
            lo = half * n_half + off
            return jnp.dot(xa, wb_ref[:, lo:lo + n_sub],
                           preferred_element_type=jnp.float32)

        def chunk_r(s):
            return _lut(RING, lax.rem(r_pos + 2 * N_DEV - 2 - s, N_DEV))

        def chunk_l(s):
            return _lut(RING, lax.rem(r_pos + 2 + s, N_DEV))

        def mk(comm, sems, s, dst):
            return pltpu.make_async_remote_copy(
                src_ref=comm.at[s], dst_ref=comm.at[s + 1],
                send_sem=sems.at[s, 0], recv_sem=sems.at[s, 1],
                device_id=(dst,), device_id_type=pl.DeviceIdType.MESH,
            )

        streams = []
        for k in range(2 * N_STREAMS):
            half = k // N_STREAMS
            off = (k % N_STREAMS) * n_sub
            dst = right if half == 0 else left
            streams.append((comms[k], semss[k], dst, half, off))

        for comm, sems, dst, half, off in streams:
            c0 = left if half == 0 else right
            comm[0] = sub_partial(c0, half, off).astype(jnp.bfloat16)
            mk(comm, sems, 0, dst).start()

        for s in range(N_DEV - 1):
            last = s == N_DEV - 2
            p_r = partial(chunk_r(s), 0)
            p_l = partial(chunk_l(s), 1)
            for comm, sems, dst, half, off in streams:
                p = p_r if half == 0 else p_l
                mk(comm, sems, s, dst).wait()
                acc = comm[s + 1].astype(jnp.float32) + p[:, off:off + n_sub]
                if not last:
                    comm[s + 1] = acc.astype(jnp.bfloat16)
                    mk(comm, sems, s + 1, dst).start()
                else:
                    col = half * n_half + off
                    out_ref[:, col:col + n_sub] = acc * jax.nn.sigmoid(acc)

        def exit_barrier(sem):
            pl.semaphore_signal(sem, inc=1, device_id=(left,),
                                device_id_type=pl.DeviceIdType.MESH)
            pl.semaphore_signal(sem, inc=1, device_id=(right,),
                                device_id_type=pl.DeviceIdType.MESH)
            pl.semaphore_wait(sem, 2)

        pl.run_scoped(exit_barrier, pltpu.SemaphoreType.REGULAR)

    sub = pltpu.VMEM((N_DEV, m_blk, n_sub), jnp.bfloat16)
    sems = pltpu.SemaphoreType.DMA((N_DEV - 1, 2))
    return pl.pallas_call(
        body,
        out_shape=jax.ShapeDtypeStruct((m_blk, n), jnp.float32),
        in_specs=[
            pl.BlockSpec(memory_space=pltpu.VMEM),
            pl.BlockSpec(memory_space=pltpu.VMEM),
        ],
        out_specs=pl.BlockSpec(memory_space=pltpu.VMEM),
        scratch_shapes=(
            [pltpu.VMEM((k_shard, n), jnp.bfloat16)]
            + [sub] * (2 * N_STREAMS)
            + [sems] * (2 * N_STREAMS)
        ),
        compiler_params=pltpu.CompilerParams(collective_id=0),
    )(x, w_mat)
